# baseline (speedup 1.0000x reference)
import jax
import jax.numpy as jnp
from jax.experimental import pallas as pl
from jax.experimental.pallas import tpu as pltpu

_YB = 16
_ZP = 40
_HALVES = 2


def _prep_w(w_ref, b_ref, nz):
    w = jnp.concatenate([w_ref[...], b_ref[...][:, :, None]], axis=2)
    w = jnp.pad(w, ((0, 0), (0, _ZP - nz), (0, 0)))
    return w.reshape(w.shape[0] * _ZP, nz + 1).astype(jnp.bfloat16)


def _body(eta_ref, xq_ref, xs_ref, wq_ref, bq_ref, ws_ref, bs_ref, out_ref):
    nz, yb, nx = xq_ref.shape
    e = wq_ref.shape[0]
    yh = yb // _HALVES
    nb = yh * nx
    wq = _prep_w(wq_ref, bq_ref, nz)
    ws = _prep_w(ws_ref, bs_ref, nz)
    ones = jnp.ones((1, nb), jnp.bfloat16)
    for h in range(_HALVES):
        ys = h * yh
        xq = jnp.concatenate(
            [xq_ref[:, ys:ys + yh, :].astype(jnp.bfloat16).reshape(nz, nb),
             ones], axis=0)
        xs = jnp.concatenate(
            [xs_ref[:, ys:ys + yh, :].astype(jnp.bfloat16).reshape(nz, nb),
             ones], axis=0)
        eta = eta_ref[ys:ys + yh, :].reshape(1, nb)

        pq = jnp.dot(wq, xq, preferred_element_type=jnp.float32)
        ps = jnp.dot(ws, xs, preferred_element_type=jnp.float32)

        accq = pq[0:nz]
        accs = ps[0:nz]
        for i in range(1, e):
            m = eta == i
            accq = jnp.where(m, pq[i * _ZP:i * _ZP + nz], accq)
            accs = jnp.where(m, ps[i * _ZP:i * _ZP + nz], accs)

        out_ref[0, :, ys:ys + yh, :] = accq.reshape(nz, yh, nx)
        out_ref[1, :, ys:ys + yh, :] = accs.reshape(nz, yh, nx)


def kernel(QT, SLI, eta, W_QT, b_QT, W_SLI, b_SLI):
    nz, ny, nx = QT.shape
    e = W_QT.shape[0]
    eta32 = eta.astype(jnp.int32)

    out = pl.pallas_call(
        _body,
        grid=(ny // _YB,),
        in_specs=[
            pl.BlockSpec((_YB, nx), lambda i: (i, 0)),
            pl.BlockSpec((nz, _YB, nx), lambda i: (0, i, 0)),
            pl.BlockSpec((nz, _YB, nx), lambda i: (0, i, 0)),
            pl.BlockSpec((e, nz, nz), lambda i: (0, 0, 0)),
            pl.BlockSpec((e, nz), lambda i: (0, 0)),
            pl.BlockSpec((e, nz, nz), lambda i: (0, 0, 0)),
            pl.BlockSpec((e, nz), lambda i: (0, 0)),
        ],
        out_specs=pl.BlockSpec((2, nz, _YB, nx), lambda i: (0, 0, i, 0)),
        out_shape=jax.ShapeDtypeStruct((2, nz, ny, nx), jnp.float32),
        compiler_params=pltpu.CompilerParams(
            dimension_semantics=("parallel",)),
    )(eta32, QT, SLI, W_QT, b_QT, W_SLI, b_SLI)
    return out


# speedup vs baseline: 1.0579x; 1.0579x over previous
import jax
import jax.numpy as jnp
from jax.experimental import pallas as pl
from jax.experimental.pallas import tpu as pltpu

_YB = 32
_ZP = 40
_HALVES = 4


def _prep_w(w_ref, b_ref, nz):
    w = jnp.concatenate([w_ref[...], b_ref[...][:, :, None]], axis=2)
    w = jnp.pad(w, ((0, 0), (0, _ZP - nz), (0, 0)))
    return w.reshape(w.shape[0] * _ZP, nz + 1).astype(jnp.bfloat16)


def _body(eta_ref, xq_ref, xs_ref, wq_ref, bq_ref, ws_ref, bs_ref, out_ref):
    nz, yb, nx = xq_ref.shape
    e = wq_ref.shape[0]
    yh = yb // _HALVES
    nb = yh * nx
    wq = _prep_w(wq_ref, bq_ref, nz)
    ws = _prep_w(ws_ref, bs_ref, nz)
    ones = jnp.ones((1, nb), jnp.bfloat16)
    for h in range(_HALVES):
        ys = h * yh
        xq = jnp.concatenate(
            [xq_ref[:, ys:ys + yh, :].astype(jnp.bfloat16).reshape(nz, nb),
             ones], axis=0)
        xs = jnp.concatenate(
            [xs_ref[:, ys:ys + yh, :].astype(jnp.bfloat16).reshape(nz, nb),
             ones], axis=0)
        eta = eta_ref[ys:ys + yh, :].reshape(1, nb)

        pq = jnp.dot(wq, xq, preferred_element_type=jnp.float32)
        ps = jnp.dot(ws, xs, preferred_element_type=jnp.float32)

        accq = pq[0:nz]
        accs = ps[0:nz]
        for i in range(1, e):
            m = eta == i
            accq = jnp.where(m, pq[i * _ZP:i * _ZP + nz], accq)
            accs = jnp.where(m, ps[i * _ZP:i * _ZP + nz], accs)

        out_ref[0, :, ys:ys + yh, :] = accq.reshape(nz, yh, nx)
        out_ref[1, :, ys:ys + yh, :] = accs.reshape(nz, yh, nx)


def kernel(QT, SLI, eta, W_QT, b_QT, W_SLI, b_SLI):
    nz, ny, nx = QT.shape
    e = W_QT.shape[0]
    eta32 = eta.astype(jnp.int32)

    out = pl.pallas_call(
        _body,
        grid=(ny // _YB,),
        in_specs=[
            pl.BlockSpec((_YB, nx), lambda i: (i, 0)),
            pl.BlockSpec((nz, _YB, nx), lambda i: (0, i, 0)),
            pl.BlockSpec((nz, _YB, nx), lambda i: (0, i, 0)),
            pl.BlockSpec((e, nz, nz), lambda i: (0, 0, 0)),
            pl.BlockSpec((e, nz), lambda i: (0, 0)),
            pl.BlockSpec((e, nz, nz), lambda i: (0, 0, 0)),
            pl.BlockSpec((e, nz), lambda i: (0, 0)),
        ],
        out_specs=pl.BlockSpec((2, nz, _YB, nx), lambda i: (0, 0, i, 0)),
        out_shape=jax.ShapeDtypeStruct((2, nz, ny, nx), jnp.float32),
        compiler_params=pltpu.CompilerParams(
            dimension_semantics=("parallel",)),
    )(eta32, QT, SLI, W_QT, b_QT, W_SLI, b_SLI)
    return out
